# Initial kernel scaffold; baseline (speedup 1.0000x reference)
import jax
import jax.numpy as jnp
from jax.experimental import pallas as pl


def kernel(nodes, edge_features, distance, edges, W_msg, b_msg, g_msg, be_msg, Wq, bq, Wk, bk, Wv, bv, Wg, bg, Wo, bo, g_upd, be_upd, sigma, beta_p):
    raise NotImplementedError("write your pallas kernel here")


# speedup vs baseline: 2.6173x; 2.6173x over previous
import functools
import math

import jax
import jax.numpy as jnp
from jax import lax
from jax.experimental import pallas as pl
from jax.experimental.pallas import tpu as pltpu
from jax.experimental.pallas import tpu_sc as plsc

NC = 2
NS = 16
LANES = 16


def _proj_body(x_ref, w1_ref, w2_ref, o1_ref, o2_ref):
    x = x_ref[0]
    o1_ref[0] = jnp.dot(x, w1_ref[...], preferred_element_type=jnp.float32)
    o2_ref[0] = jnp.dot(x, w2_ref[...], preferred_element_type=jnp.float32)


def _proj(nodes, w1, w2):
    B, N, F = nodes.shape
    RP = 512
    grid = (B, N // RP)
    out = jax.ShapeDtypeStruct((B, N, F), jnp.float32)
    return pl.pallas_call(
        _proj_body,
        grid=grid,
        in_specs=[
            pl.BlockSpec((1, RP, F), lambda b, i: (b, i, 0)),
            pl.BlockSpec((F, F), lambda b, i: (0, 0)),
            pl.BlockSpec((F, F), lambda b, i: (0, 0)),
        ],
        out_specs=[
            pl.BlockSpec((1, RP, F), lambda b, i: (b, i, 0)),
            pl.BlockSpec((1, RP, F), lambda b, i: (b, i, 0)),
        ],
        out_shape=[out, out],
    )(nodes, w1, w2)


def _make_gather(BE, F, C=128):
    n_w = NC * NS
    per_w = BE // n_w
    n_chunks = per_w // C
    mesh = plsc.VectorSubcoreMesh(core_axis_name="c", subcore_axis_name="s")

    def body(p1, p2, i0, i1, t, i0_v, i1_v, r0_v, r1_v, sem0, sem1):
        wid = lax.axis_index("s") * NC + lax.axis_index("c")
        base = wid * per_w

        def chunk(i, carry):
            off = base + i * C
            pltpu.sync_copy(i0.at[pl.ds(off, C)], i0_v)
            pltpu.sync_copy(i1.at[pl.ds(off, C)], i1_v)
            cp0 = pltpu.async_copy(p1.at[i0_v], r0_v, sem0)
            cp1 = pltpu.async_copy(p2.at[i1_v], r1_v, sem1)
            cp0.wait()
            cp1.wait()

            def add_row(r, c2):
                for j in range(F // LANES):
                    sl = pl.ds(j * LANES, LANES)
                    r0_v[r, sl] = r0_v[r, sl] + r1_v[r, sl]
                return c2

            lax.fori_loop(0, C, add_row, 0)
            pltpu.sync_copy(r0_v, t.at[pl.ds(off, C)])
            return carry

        lax.fori_loop(0, n_chunks, chunk, 0)

    return pl.kernel(
        body,
        mesh=mesh,
        compiler_params=pltpu.CompilerParams(needs_layout_passes=False),
        out_type=jax.ShapeDtypeStruct((BE, F), jnp.float32),
        scratch_types=[
            pltpu.VMEM((C,), jnp.int32),
            pltpu.VMEM((C,), jnp.int32),
            pltpu.VMEM((C, F), jnp.float32),
            pltpu.VMEM((C, F), jnp.float32),
            pltpu.SemaphoreType.DMA,
            pltpu.SemaphoreType.DMA,
        ],
    )


def _make_scatter(B, E, N, F, C=256):
    bpc = B // NC
    rows = bpc * N
    n_edges = bpc * E
    n_chunks = n_edges // C
    blk = LANES * C
    acc_sz = LANES * rows
    assert F == NS * LANES
    mesh = plsc.VectorSubcoreMesh(core_axis_name="c", subcore_axis_name="s")

    def body(wm1d, pos1d, agg1d, pos_v, wm_v, acc_v):
        c = lax.axis_index("c")
        s = lax.axis_index("s")
        iota = lax.iota(jnp.int32, LANES)
        zeros = jnp.zeros((LANES,), jnp.float32)

        def zrow(i, carry):
            acc_v[pl.ds(i * LANES, LANES)] = zeros
            return carry

        lax.fori_loop(0, acc_sz // LANES, zrow, 0)

        def chunk(i, carry):
            g = c * n_chunks + i
            pltpu.sync_copy(pos1d.at[pl.ds(g * C * LANES, C * LANES)], pos_v)
            pltpu.sync_copy(wm1d.at[pl.ds((g * NS + s) * blk, blk)], wm_v)
            for e in range(C):
                pos = pos_v[pl.ds(e * LANES, LANES)]
                val = plsc.load_gather(wm_v, [iota * C + e])
                plsc.addupdate_scatter(acc_v, [pos], val)
            return carry

        lax.fori_loop(0, n_chunks, chunk, 0)
        pltpu.sync_copy(acc_v,
                        agg1d.at[pl.ds((c * NS + s) * acc_sz, acc_sz)])

    return pl.kernel(
        body,
        mesh=mesh,
        compiler_params=pltpu.CompilerParams(needs_layout_passes=False),
        out_type=jax.ShapeDtypeStruct((NC * NS * acc_sz,), jnp.float32),
        scratch_types=[
            pltpu.VMEM((C * LANES,), jnp.int32),
            pltpu.VMEM((blk,), jnp.float32),
            pltpu.VMEM((acc_sz,), jnp.float32),
        ],
    )


def _gelu(x):
    return 0.5 * x * (1.0 + lax.erf(x * (1.0 / math.sqrt(2.0))))


def _edge_body(t_ref, ef_ref, d_ref, w3_ref, bm_ref, g_ref, be_ref, sig_ref,
               beta_ref, o_ref, ot_ref):
    u = t_ref[...] + jnp.dot(ef_ref[...], w3_ref[...],
                             preferred_element_type=jnp.float32) + bm_ref[...]
    g = _gelu(u)
    mu = jnp.mean(g, axis=-1, keepdims=True)
    xc = g - mu
    var = jnp.mean(xc * xc, axis=-1, keepdims=True)
    m = xc * lax.rsqrt(var + 1e-3) * g_ref[...] + be_ref[...]
    sig = sig_ref[0, 0]
    beta = beta_ref[0, 0]
    d = d_ref[...]
    xw = (d * d) * (0.5 / (sig * sig))
    w = jnp.exp(-jnp.exp(beta * jnp.log(xw)))
    wm = m * w
    o_ref[...] = wm
    R, F = wm.shape
    CC = 256
    for cc in range(R // CC):
        mcc = wm[cc * CC:(cc + 1) * CC, :].T
        ot_ref[pl.ds(cc * CC * F, CC * F)] = mcc.reshape(CC * F)


def _edge_stage(t, ef, dist, w3, bm, gm, bem, sig, beta):
    BE, F = t.shape
    DE = ef.shape[-1]
    R = 1024
    grid = (BE // R,)
    smem = functools.partial(pl.BlockSpec, memory_space=pltpu.SMEM)
    return pl.pallas_call(
        _edge_body,
        grid=grid,
        in_specs=[
            pl.BlockSpec((R, F), lambda i: (i, 0)),
            pl.BlockSpec((R, DE), lambda i: (i, 0)),
            pl.BlockSpec((R, 1), lambda i: (i, 0)),
            pl.BlockSpec((DE, F), lambda i: (0, 0)),
            pl.BlockSpec((1, F), lambda i: (0, 0)),
            pl.BlockSpec((1, F), lambda i: (0, 0)),
            pl.BlockSpec((1, F), lambda i: (0, 0)),
            smem((1, 1), lambda i: (0, 0)),
            smem((1, 1), lambda i: (0, 0)),
        ],
        out_specs=[
            pl.BlockSpec((R, F), lambda i: (i, 0)),
            pl.BlockSpec((R * F,), lambda i: (i,)),
        ],
        out_shape=[
            jax.ShapeDtypeStruct((BE, F), jnp.float32),
            jax.ShapeDtypeStruct((BE * F,), jnp.float32),
        ],
    )(t, ef, dist, w3, bm, gm, bem, sig, beta)


def _qkvg_body(n_ref, at_ref, wt_ref, wb_ref, b_ref, oq_ref, ok_ref, ov_ref,
               og_ref):
    F = oq_ref.shape[-1]
    y = jnp.dot(n_ref[0], wt_ref[...], preferred_element_type=jnp.float32)
    y += lax.dot_general(at_ref[...], wb_ref[...], (((0,), (0,)), ((), ())),
                         preferred_element_type=jnp.float32)
    y += b_ref[...]
    oq_ref[0] = y[:, :F]
    ok_ref[0] = y[:, F:2 * F]
    ov_ref[0] = y[:, 2 * F:3 * F]
    og_ref[0] = jax.nn.sigmoid(y[:, 3 * F:])


def _qkvg(nodes, agg_t, wtop, wbot, bcat):
    B, N, F = nodes.shape
    RP = 512
    grid = (B, N // RP)
    out = jax.ShapeDtypeStruct((B, N, F), jnp.float32)
    nrp = N // RP
    return pl.pallas_call(
        _qkvg_body,
        grid=grid,
        in_specs=[
            pl.BlockSpec((1, RP, F), lambda b, i: (b, i, 0)),
            pl.BlockSpec((F, RP), lambda b, i: (0, b * nrp + i)),
            pl.BlockSpec((F, 4 * F), lambda b, i: (0, 0)),
            pl.BlockSpec((F, 4 * F), lambda b, i: (0, 0)),
            pl.BlockSpec((1, 4 * F), lambda b, i: (0, 0)),
        ],
        out_specs=[
            pl.BlockSpec((1, RP, F), lambda b, i: (b, i, 0)),
            pl.BlockSpec((1, RP, F), lambda b, i: (b, i, 0)),
            pl.BlockSpec((1, RP, F), lambda b, i: (b, i, 0)),
            pl.BlockSpec((1, RP, F), lambda b, i: (b, i, 0)),
        ],
        out_shape=[out, out, out, out],
    )(nodes, agg_t, wtop, wbot, bcat)


def _attn_body(q_ref, k_ref, v_ref, g_ref, wo_ref, bo_ref, gu_ref, bu_ref,
               o_ref, *, H):
    q = q_ref[0]
    k = k_ref[0]
    v = v_ref[0]
    gt = g_ref[0]
    F = q.shape[-1]
    dh = F // H
    scale = 1.0 / math.sqrt(dh)
    outs = []
    for h in range(H):
        qh = q[:, h * dh:(h + 1) * dh]
        kh = k[:, h * dh:(h + 1) * dh]
        vh = v[:, h * dh:(h + 1) * dh]
        s = lax.dot_general(qh, kh, (((1,), (1,)), ((), ())),
                            preferred_element_type=jnp.float32) * scale
        mx = jnp.max(s, axis=-1, keepdims=True)
        p = jnp.exp(s - mx)
        wgt = p / jnp.sum(p, axis=-1, keepdims=True)
        outs.append(jnp.dot(wgt, vh, preferred_element_type=jnp.float32))
    att = jnp.concatenate(outs, axis=1) * gt
    out = jnp.dot(att, wo_ref[...], preferred_element_type=jnp.float32) + bo_ref[...]
    out = _gelu(out)
    mu = jnp.mean(out, axis=-1, keepdims=True)
    xc = out - mu
    var = jnp.mean(xc * xc, axis=-1, keepdims=True)
    o_ref[0] = xc * lax.rsqrt(var + 1e-3) * gu_ref[...] + bu_ref[...]


def _attention(q, k, v, gt, wo, bo, gu, bu, H):
    B, N, F = q.shape
    QB = 256
    grid = (B, N // QB)
    return pl.pallas_call(
        functools.partial(_attn_body, H=H),
        grid=grid,
        in_specs=[
            pl.BlockSpec((1, QB, F), lambda b, i: (b, i, 0)),
            pl.BlockSpec((1, N, F), lambda b, i: (b, 0, 0)),
            pl.BlockSpec((1, N, F), lambda b, i: (b, 0, 0)),
            pl.BlockSpec((1, QB, F), lambda b, i: (b, i, 0)),
            pl.BlockSpec((F, F), lambda b, i: (0, 0)),
            pl.BlockSpec((1, F), lambda b, i: (0, 0)),
            pl.BlockSpec((1, F), lambda b, i: (0, 0)),
            pl.BlockSpec((1, F), lambda b, i: (0, 0)),
        ],
        out_specs=pl.BlockSpec((1, QB, F), lambda b, i: (b, i, 0)),
        out_shape=jax.ShapeDtypeStruct((B, N, F), jnp.float32),
    )(q, k, v, gt, wo, bo, gu, bu)


def kernel(nodes, edge_features, distance, edges, W_msg, b_msg, g_msg, be_msg,
           Wq, bq, Wk, bk, Wv, bv, Wg, bg, Wo, bo, g_upd, be_upd, sigma,
           beta_p):
    B, N, F = nodes.shape
    E = edges.shape[1]
    DE = edge_features.shape[-1]
    H = 8
    BE = B * E

    w1 = W_msg[:F]
    w2 = W_msg[F:2 * F]
    w3 = W_msg[2 * F:]

    p1, p2 = _proj(nodes, w1, w2)

    e0 = edges[..., 0].astype(jnp.int32)
    e1 = edges[..., 1].astype(jnp.int32)
    boff = (jnp.arange(B, dtype=jnp.int32) * N)[:, None]
    idx0 = (e0 + boff).reshape(BE)
    idx1 = (e1 + boff).reshape(BE)
    t = _make_gather(BE, F)(p1.reshape(B * N, F), p2.reshape(B * N, F),
                            idx0, idx1)

    wm, wm_t = _edge_stage(t, edge_features.reshape(BE, DE),
                     distance.reshape(BE, 1), w3,
                     b_msg.reshape(1, F), g_msg.reshape(1, F),
                     be_msg.reshape(1, F), sigma.reshape(1, 1),
                     beta_p.reshape(1, 1))

    bpc = B // NC
    locoff = ((jnp.arange(B, dtype=jnp.int32) % bpc) * N)[:, None]
    idx_sc = (e1 + locoff).reshape(BE)
    lane_off = jnp.arange(LANES, dtype=jnp.int32) * (bpc * N)
    pos1d = (idx_sc[:, None] + lane_off[None, :]).reshape(BE * LANES)
    agg1d = _make_scatter(B, E, N, F)(wm_t, pos1d)
    agg_t = agg1d.reshape(NC, NS, LANES, bpc * N).transpose(1, 2, 0, 3)
    agg_t = agg_t.reshape(F, B * N)

    wcat = jnp.concatenate([Wq, Wk, Wv, Wg], axis=1)
    bcat = jnp.concatenate([bq, bk, bv, bg]).reshape(1, 4 * F)
    q, k, v, gt = _qkvg(nodes, agg_t, wcat[:F], wcat[F:], bcat)
    updated = _attention(q, k, v, gt, Wo, bo.reshape(1, F),
                         g_upd.reshape(1, F), be_upd.reshape(1, F), H)

    return (updated, wm.reshape(B, E, F), distance, edges)


# speedup vs baseline: 2.9462x; 1.1257x over previous
import functools
import math

import jax
import jax.numpy as jnp
from jax import lax
from jax.experimental import pallas as pl
from jax.experimental.pallas import tpu as pltpu
from jax.experimental.pallas import tpu_sc as plsc

NC = 2
NS = 16
LANES = 16


def _proj_body(x_ref, w1_ref, w2_ref, o1_ref, o2_ref):
    x = x_ref[0]
    o1_ref[0] = jnp.dot(x, w1_ref[...], preferred_element_type=jnp.float32)
    o2_ref[0] = jnp.dot(x, w2_ref[...], preferred_element_type=jnp.float32)


def _proj(nodes, w1, w2):
    B, N, F = nodes.shape
    RP = 512
    grid = (B, N // RP)
    out = jax.ShapeDtypeStruct((B, N, F), jnp.float32)
    return pl.pallas_call(
        _proj_body,
        grid=grid,
        in_specs=[
            pl.BlockSpec((1, RP, F), lambda b, i: (b, i, 0)),
            pl.BlockSpec((F, F), lambda b, i: (0, 0)),
            pl.BlockSpec((F, F), lambda b, i: (0, 0)),
        ],
        out_specs=[
            pl.BlockSpec((1, RP, F), lambda b, i: (b, i, 0)),
            pl.BlockSpec((1, RP, F), lambda b, i: (b, i, 0)),
        ],
        out_shape=[out, out],
    )(nodes, w1, w2)


def _make_gather(BE, F, C=128):
    n_w = NC * NS
    per_w = BE // n_w
    n_chunks = per_w // C
    mesh = plsc.VectorSubcoreMesh(core_axis_name="c", subcore_axis_name="s")

    def body(p1, p2, i0, i1, t, i0_v, i1_v, r0_v, r1_v, sem0, sem1):
        wid = lax.axis_index("s") * NC + lax.axis_index("c")
        base = wid * per_w

        def chunk(i, carry):
            off = base + i * C
            pltpu.sync_copy(i0.at[pl.ds(off, C)], i0_v)
            pltpu.sync_copy(i1.at[pl.ds(off, C)], i1_v)
            cp0 = pltpu.async_copy(p1.at[i0_v], r0_v, sem0)
            cp1 = pltpu.async_copy(p2.at[i1_v], r1_v, sem1)
            cp0.wait()
            cp1.wait()

            def add_row(r, c2):
                for j in range(F // LANES):
                    sl = pl.ds(j * LANES, LANES)
                    r0_v[r, sl] = r0_v[r, sl] + r1_v[r, sl]
                return c2

            lax.fori_loop(0, C, add_row, 0)
            pltpu.sync_copy(r0_v, t.at[pl.ds(off, C)])
            return carry

        lax.fori_loop(0, n_chunks, chunk, 0)

    return pl.kernel(
        body,
        mesh=mesh,
        compiler_params=pltpu.CompilerParams(needs_layout_passes=False),
        out_type=jax.ShapeDtypeStruct((BE, F), jnp.float32),
        scratch_types=[
            pltpu.VMEM((C,), jnp.int32),
            pltpu.VMEM((C,), jnp.int32),
            pltpu.VMEM((C, F), jnp.float32),
            pltpu.VMEM((C, F), jnp.float32),
            pltpu.SemaphoreType.DMA,
            pltpu.SemaphoreType.DMA,
        ],
    )


def _make_scatter(B, E, N, F, C=256):
    bpc = B // NC
    rows = bpc * N
    n_edges = bpc * E
    n_chunks = n_edges // C
    blk = LANES * C
    acc_sz = LANES * rows
    assert F == NS * LANES
    mesh = plsc.VectorSubcoreMesh(core_axis_name="c", subcore_axis_name="s")

    def body(wm1d, pos1d, agg1d, pos_v0, pos_v1, wm_v0, wm_v1, acc_v, sem0,
             sem1):
        c = lax.axis_index("c")
        s = lax.axis_index("s")
        iota = lax.iota(jnp.int32, LANES)
        zeros = jnp.zeros((LANES,), jnp.float32)
        pos_bufs = (pos_v0, pos_v1)
        wm_bufs = (wm_v0, wm_v1)
        sems = (sem0, sem1)

        def zrow(i, carry):
            acc_v[pl.ds(i * LANES, LANES)] = zeros
            return carry

        lax.fori_loop(0, acc_sz // LANES, zrow, 0)

        def issue(g, b):
            pltpu.async_copy(pos1d.at[pl.ds(g * C * LANES, C * LANES)],
                             pos_bufs[b], sems[b])
            pltpu.async_copy(wm1d.at[pl.ds((g * NS + s) * blk, blk)],
                             wm_bufs[b], sems[b])

        def drain(g, b):
            pltpu.make_async_copy(pos1d.at[pl.ds(g * C * LANES, C * LANES)],
                                  pos_bufs[b], sems[b]).wait()
            pltpu.make_async_copy(wm1d.at[pl.ds((g * NS + s) * blk, blk)],
                                  wm_bufs[b], sems[b]).wait()

        g0 = c * n_chunks
        issue(g0, 0)

        def outer(i, carry):
            for b in range(2):
                li = 2 * i + b
                g = g0 + li
                drain(g, b)

                @pl.when(li + 1 < n_chunks)
                def _():
                    issue(g + 1, b ^ 1)

                pv = pos_bufs[b]
                wv = wm_bufs[b]
                for e in range(C):
                    pos = pv[pl.ds(e * LANES, LANES)]
                    val = plsc.load_gather(wv, [iota * C + e])
                    plsc.addupdate_scatter(acc_v, [pos], val)
            return carry

        lax.fori_loop(0, n_chunks // 2, outer, 0)
        pltpu.sync_copy(acc_v,
                        agg1d.at[pl.ds((c * NS + s) * acc_sz, acc_sz)])

    return pl.kernel(
        body,
        mesh=mesh,
        compiler_params=pltpu.CompilerParams(needs_layout_passes=False),
        out_type=jax.ShapeDtypeStruct((NC * NS * acc_sz,), jnp.float32),
        scratch_types=[
            pltpu.VMEM((C * LANES,), jnp.int32),
            pltpu.VMEM((C * LANES,), jnp.int32),
            pltpu.VMEM((blk,), jnp.float32),
            pltpu.VMEM((blk,), jnp.float32),
            pltpu.VMEM((acc_sz,), jnp.float32),
            pltpu.SemaphoreType.DMA,
            pltpu.SemaphoreType.DMA,
        ],
    )


def _gelu(x):
    return 0.5 * x * (1.0 + lax.erf(x * (1.0 / math.sqrt(2.0))))


def _edge_body(t_ref, ef_ref, d_ref, w3_ref, bm_ref, g_ref, be_ref, sig_ref,
               beta_ref, o_ref, ot_ref):
    u = t_ref[...] + jnp.dot(ef_ref[...], w3_ref[...],
                             preferred_element_type=jnp.float32) + bm_ref[...]
    g = _gelu(u)
    mu = jnp.mean(g, axis=-1, keepdims=True)
    xc = g - mu
    var = jnp.mean(xc * xc, axis=-1, keepdims=True)
    m = xc * lax.rsqrt(var + 1e-3) * g_ref[...] + be_ref[...]
    sig = sig_ref[0, 0]
    beta = beta_ref[0, 0]
    d = d_ref[...]
    xw = (d * d) * (0.5 / (sig * sig))
    w = jnp.exp(-jnp.exp(beta * jnp.log(xw)))
    wm = m * w
    o_ref[...] = wm
    R, F = wm.shape
    CC = 256
    for cc in range(R // CC):
        mcc = wm[cc * CC:(cc + 1) * CC, :].T
        ot_ref[pl.ds(cc * CC * F, CC * F)] = mcc.reshape(CC * F)


def _edge_stage(t, ef, dist, w3, bm, gm, bem, sig, beta):
    BE, F = t.shape
    DE = ef.shape[-1]
    R = 1024
    grid = (BE // R,)
    smem = functools.partial(pl.BlockSpec, memory_space=pltpu.SMEM)
    return pl.pallas_call(
        _edge_body,
        grid=grid,
        in_specs=[
            pl.BlockSpec((R, F), lambda i: (i, 0)),
            pl.BlockSpec((R, DE), lambda i: (i, 0)),
            pl.BlockSpec((R, 1), lambda i: (i, 0)),
            pl.BlockSpec((DE, F), lambda i: (0, 0)),
            pl.BlockSpec((1, F), lambda i: (0, 0)),
            pl.BlockSpec((1, F), lambda i: (0, 0)),
            pl.BlockSpec((1, F), lambda i: (0, 0)),
            smem((1, 1), lambda i: (0, 0)),
            smem((1, 1), lambda i: (0, 0)),
        ],
        out_specs=[
            pl.BlockSpec((R, F), lambda i: (i, 0)),
            pl.BlockSpec((R * F,), lambda i: (i,)),
        ],
        out_shape=[
            jax.ShapeDtypeStruct((BE, F), jnp.float32),
            jax.ShapeDtypeStruct((BE * F,), jnp.float32),
        ],
    )(t, ef, dist, w3, bm, gm, bem, sig, beta)


def _qkvg_body(n_ref, at_ref, wt_ref, wb_ref, b_ref, oq_ref, ok_ref, ov_ref,
               og_ref):
    F = oq_ref.shape[-1]
    y = jnp.dot(n_ref[0], wt_ref[...], preferred_element_type=jnp.float32)
    y += lax.dot_general(at_ref[...], wb_ref[...], (((0,), (0,)), ((), ())),
                         preferred_element_type=jnp.float32)
    y += b_ref[...]
    oq_ref[0] = y[:, :F]
    ok_ref[0] = y[:, F:2 * F]
    ov_ref[0] = y[:, 2 * F:3 * F]
    og_ref[0] = jax.nn.sigmoid(y[:, 3 * F:])


def _qkvg(nodes, agg_t, wtop, wbot, bcat):
    B, N, F = nodes.shape
    RP = 512
    grid = (B, N // RP)
    out = jax.ShapeDtypeStruct((B, N, F), jnp.float32)
    nrp = N // RP
    return pl.pallas_call(
        _qkvg_body,
        grid=grid,
        in_specs=[
            pl.BlockSpec((1, RP, F), lambda b, i: (b, i, 0)),
            pl.BlockSpec((F, RP), lambda b, i: (0, b * nrp + i)),
            pl.BlockSpec((F, 4 * F), lambda b, i: (0, 0)),
            pl.BlockSpec((F, 4 * F), lambda b, i: (0, 0)),
            pl.BlockSpec((1, 4 * F), lambda b, i: (0, 0)),
        ],
        out_specs=[
            pl.BlockSpec((1, RP, F), lambda b, i: (b, i, 0)),
            pl.BlockSpec((1, RP, F), lambda b, i: (b, i, 0)),
            pl.BlockSpec((1, RP, F), lambda b, i: (b, i, 0)),
            pl.BlockSpec((1, RP, F), lambda b, i: (b, i, 0)),
        ],
        out_shape=[out, out, out, out],
    )(nodes, agg_t, wtop, wbot, bcat)


def _attn_body(q_ref, k_ref, v_ref, g_ref, wo_ref, bo_ref, gu_ref, bu_ref,
               o_ref, *, H):
    q = q_ref[0]
    k = k_ref[0]
    v = v_ref[0]
    gt = g_ref[0]
    F = q.shape[-1]
    dh = F // H
    scale = 1.0 / math.sqrt(dh)
    outs = []
    for h in range(H):
        qh = q[:, h * dh:(h + 1) * dh]
        kh = k[:, h * dh:(h + 1) * dh]
        vh = v[:, h * dh:(h + 1) * dh]
        s = lax.dot_general(qh, kh, (((1,), (1,)), ((), ())),
                            preferred_element_type=jnp.float32) * scale
        mx = jnp.max(s, axis=-1, keepdims=True)
        p = jnp.exp(s - mx)
        wgt = p / jnp.sum(p, axis=-1, keepdims=True)
        outs.append(jnp.dot(wgt, vh, preferred_element_type=jnp.float32))
    att = jnp.concatenate(outs, axis=1) * gt
    out = jnp.dot(att, wo_ref[...], preferred_element_type=jnp.float32) + bo_ref[...]
    out = _gelu(out)
    mu = jnp.mean(out, axis=-1, keepdims=True)
    xc = out - mu
    var = jnp.mean(xc * xc, axis=-1, keepdims=True)
    o_ref[0] = xc * lax.rsqrt(var + 1e-3) * gu_ref[...] + bu_ref[...]


def _attention(q, k, v, gt, wo, bo, gu, bu, H):
    B, N, F = q.shape
    QB = 256
    grid = (B, N // QB)
    return pl.pallas_call(
        functools.partial(_attn_body, H=H),
        grid=grid,
        in_specs=[
            pl.BlockSpec((1, QB, F), lambda b, i: (b, i, 0)),
            pl.BlockSpec((1, N, F), lambda b, i: (b, 0, 0)),
            pl.BlockSpec((1, N, F), lambda b, i: (b, 0, 0)),
            pl.BlockSpec((1, QB, F), lambda b, i: (b, i, 0)),
            pl.BlockSpec((F, F), lambda b, i: (0, 0)),
            pl.BlockSpec((1, F), lambda b, i: (0, 0)),
            pl.BlockSpec((1, F), lambda b, i: (0, 0)),
            pl.BlockSpec((1, F), lambda b, i: (0, 0)),
        ],
        out_specs=pl.BlockSpec((1, QB, F), lambda b, i: (b, i, 0)),
        out_shape=jax.ShapeDtypeStruct((B, N, F), jnp.float32),
    )(q, k, v, gt, wo, bo, gu, bu)


def kernel(nodes, edge_features, distance, edges, W_msg, b_msg, g_msg, be_msg,
           Wq, bq, Wk, bk, Wv, bv, Wg, bg, Wo, bo, g_upd, be_upd, sigma,
           beta_p):
    B, N, F = nodes.shape
    E = edges.shape[1]
    DE = edge_features.shape[-1]
    H = 8
    BE = B * E

    w1 = W_msg[:F]
    w2 = W_msg[F:2 * F]
    w3 = W_msg[2 * F:]

    p1, p2 = _proj(nodes, w1, w2)

    e0 = edges[..., 0].astype(jnp.int32)
    e1 = edges[..., 1].astype(jnp.int32)
    boff = (jnp.arange(B, dtype=jnp.int32) * N)[:, None]
    idx0 = (e0 + boff).reshape(BE)
    idx1 = (e1 + boff).reshape(BE)
    t = _make_gather(BE, F)(p1.reshape(B * N, F), p2.reshape(B * N, F),
                            idx0, idx1)

    wm, wm_t = _edge_stage(t, edge_features.reshape(BE, DE),
                     distance.reshape(BE, 1), w3,
                     b_msg.reshape(1, F), g_msg.reshape(1, F),
                     be_msg.reshape(1, F), sigma.reshape(1, 1),
                     beta_p.reshape(1, 1))

    bpc = B // NC
    locoff = ((jnp.arange(B, dtype=jnp.int32) % bpc) * N)[:, None]
    idx_sc = (e1 + locoff).reshape(BE)
    lane_off = jnp.arange(LANES, dtype=jnp.int32) * (bpc * N)
    pos1d = (idx_sc[:, None] + lane_off[None, :]).reshape(BE * LANES)
    agg1d = _make_scatter(B, E, N, F)(wm_t, pos1d)
    agg_t = agg1d.reshape(NC, NS, LANES, bpc * N).transpose(1, 2, 0, 3)
    agg_t = agg_t.reshape(F, B * N)

    wcat = jnp.concatenate([Wq, Wk, Wv, Wg], axis=1)
    bcat = jnp.concatenate([bq, bk, bv, bg]).reshape(1, 4 * F)
    q, k, v, gt = _qkvg(nodes, agg_t, wcat[:F], wcat[F:], bcat)
    updated = _attention(q, k, v, gt, Wo, bo.reshape(1, F),
                         g_upd.reshape(1, F), be_upd.reshape(1, F), H)

    return (updated, wm.reshape(B, E, F), distance, edges)


# speedup vs baseline: 3.5645x; 1.2099x over previous
import functools
import math

import jax
import jax.numpy as jnp
from jax import lax
from jax.experimental import pallas as pl
from jax.experimental.pallas import tpu as pltpu
from jax.experimental.pallas import tpu_sc as plsc

NC = 2
NS = 16
LANES = 16


def _proj_body(x_ref, w1_ref, w2_ref, o1_ref, o2_ref):
    x = x_ref[0]
    o1_ref[0] = jnp.dot(x, w1_ref[...], preferred_element_type=jnp.float32)
    o2_ref[0] = jnp.dot(x, w2_ref[...], preferred_element_type=jnp.float32)


def _proj(nodes, w1, w2):
    B, N, F = nodes.shape
    RP = 512
    grid = (B, N // RP)
    out = jax.ShapeDtypeStruct((B, N, F), jnp.float32)
    return pl.pallas_call(
        _proj_body,
        grid=grid,
        in_specs=[
            pl.BlockSpec((1, RP, F), lambda b, i: (b, i, 0)),
            pl.BlockSpec((F, F), lambda b, i: (0, 0)),
            pl.BlockSpec((F, F), lambda b, i: (0, 0)),
        ],
        out_specs=[
            pl.BlockSpec((1, RP, F), lambda b, i: (b, i, 0)),
            pl.BlockSpec((1, RP, F), lambda b, i: (b, i, 0)),
        ],
        out_shape=[out, out],
    )(nodes, w1, w2)


def _make_gather(BE, F, C=128):
    n_w = NC * NS
    per_w = BE // n_w
    n_chunks = per_w // C
    mesh = plsc.VectorSubcoreMesh(core_axis_name="c", subcore_axis_name="s")

    def body(p1, p2, i0, i1, t, i0_v, i1_v, r0_v, r1_v, sem0, sem1):
        wid = lax.axis_index("s") * NC + lax.axis_index("c")
        base = wid * per_w

        def chunk(i, carry):
            off = base + i * C
            pltpu.sync_copy(i0.at[pl.ds(off, C)], i0_v)
            pltpu.sync_copy(i1.at[pl.ds(off, C)], i1_v)
            cp0 = pltpu.async_copy(p1.at[i0_v], r0_v, sem0)
            cp1 = pltpu.async_copy(p2.at[i1_v], r1_v, sem1)
            cp0.wait()
            cp1.wait()

            def add_row(r, c2):
                for j in range(F // LANES):
                    sl = pl.ds(j * LANES, LANES)
                    r0_v[r, sl] = r0_v[r, sl] + r1_v[r, sl]
                return c2

            lax.fori_loop(0, C, add_row, 0)
            pltpu.sync_copy(r0_v, t.at[pl.ds(off, C)])
            return carry

        lax.fori_loop(0, n_chunks, chunk, 0)

    return pl.kernel(
        body,
        mesh=mesh,
        compiler_params=pltpu.CompilerParams(needs_layout_passes=False),
        out_type=jax.ShapeDtypeStruct((BE, F), jnp.float32),
        scratch_types=[
            pltpu.VMEM((C,), jnp.int32),
            pltpu.VMEM((C,), jnp.int32),
            pltpu.VMEM((C, F), jnp.float32),
            pltpu.VMEM((C, F), jnp.float32),
            pltpu.SemaphoreType.DMA,
            pltpu.SemaphoreType.DMA,
        ],
    )


def _make_scatter(B, E, N, F, C=256):
    bpc = B // NC
    rows = bpc * N
    n_edges = bpc * E
    n_chunks = n_edges // C
    blk = LANES * C
    acc_sz = LANES * rows
    assert F == NS * LANES
    mesh = plsc.VectorSubcoreMesh(core_axis_name="c", subcore_axis_name="s")

    def body(wm1d, pos1d, agg1d, pos_v0, pos_v1, wm_v0, wm_v1, acc_v, sem0,
             sem1):
        c = lax.axis_index("c")
        s = lax.axis_index("s")
        iota = lax.iota(jnp.int32, LANES)
        zeros = jnp.zeros((LANES,), jnp.float32)
        pos_bufs = (pos_v0, pos_v1)
        wm_bufs = (wm_v0, wm_v1)
        sems = (sem0, sem1)

        @plsc.parallel_loop(0, acc_sz // LANES, unroll=8)
        def _zrow(i):
            acc_v[pl.ds(i * LANES, LANES)] = zeros

        def issue(g, b):
            pltpu.async_copy(pos1d.at[pl.ds(g * C * LANES, C * LANES)],
                             pos_bufs[b], sems[b])
            pltpu.async_copy(wm1d.at[pl.ds((g * NS + s) * blk, blk)],
                             wm_bufs[b], sems[b])

        def drain(g, b):
            pltpu.make_async_copy(pos1d.at[pl.ds(g * C * LANES, C * LANES)],
                                  pos_bufs[b], sems[b]).wait()
            pltpu.make_async_copy(wm1d.at[pl.ds((g * NS + s) * blk, blk)],
                                  wm_bufs[b], sems[b]).wait()

        g0 = c * n_chunks
        issue(g0, 0)

        def outer(i, carry):
            for b in range(2):
                li = 2 * i + b
                g = g0 + li
                drain(g, b)

                @pl.when(li + 1 < n_chunks)
                def _():
                    issue(g + 1, b ^ 1)

                pv = pos_bufs[b]
                wv = wm_bufs[b]

                @plsc.parallel_loop(0, C, unroll=8)
                def _edge(e):
                    pos = pv[pl.ds(e * LANES, LANES)]
                    val = plsc.load_gather(wv, [iota * C + e])
                    plsc.addupdate_scatter(acc_v, [pos], val)
            return carry

        lax.fori_loop(0, n_chunks // 2, outer, 0)
        pltpu.sync_copy(acc_v,
                        agg1d.at[pl.ds((c * NS + s) * acc_sz, acc_sz)])

    return pl.kernel(
        body,
        mesh=mesh,
        compiler_params=pltpu.CompilerParams(needs_layout_passes=False),
        out_type=jax.ShapeDtypeStruct((NC * NS * acc_sz,), jnp.float32),
        scratch_types=[
            pltpu.VMEM((C * LANES,), jnp.int32),
            pltpu.VMEM((C * LANES,), jnp.int32),
            pltpu.VMEM((blk,), jnp.float32),
            pltpu.VMEM((blk,), jnp.float32),
            pltpu.VMEM((acc_sz,), jnp.float32),
            pltpu.SemaphoreType.DMA,
            pltpu.SemaphoreType.DMA,
        ],
    )


def _gelu(x):
    return 0.5 * x * (1.0 + lax.erf(x * (1.0 / math.sqrt(2.0))))


def _edge_body(t_ref, ef_ref, d_ref, w3_ref, bm_ref, g_ref, be_ref, sig_ref,
               beta_ref, o_ref, ot_ref):
    u = t_ref[...] + jnp.dot(ef_ref[...], w3_ref[...],
                             preferred_element_type=jnp.float32) + bm_ref[...]
    g = _gelu(u)
    mu = jnp.mean(g, axis=-1, keepdims=True)
    xc = g - mu
    var = jnp.mean(xc * xc, axis=-1, keepdims=True)
    m = xc * lax.rsqrt(var + 1e-3) * g_ref[...] + be_ref[...]
    sig = sig_ref[0, 0]
    beta = beta_ref[0, 0]
    d = d_ref[...]
    xw = (d * d) * (0.5 / (sig * sig))
    w = jnp.exp(-jnp.exp(beta * jnp.log(xw)))
    wm = m * w
    o_ref[...] = wm
    R, F = wm.shape
    CC = 256
    for cc in range(R // CC):
        mcc = wm[cc * CC:(cc + 1) * CC, :].T
        ot_ref[pl.ds(cc * CC * F, CC * F)] = mcc.reshape(CC * F)


def _edge_stage(t, ef, dist, w3, bm, gm, bem, sig, beta):
    BE, F = t.shape
    DE = ef.shape[-1]
    R = 1024
    grid = (BE // R,)
    smem = functools.partial(pl.BlockSpec, memory_space=pltpu.SMEM)
    return pl.pallas_call(
        _edge_body,
        grid=grid,
        in_specs=[
            pl.BlockSpec((R, F), lambda i: (i, 0)),
            pl.BlockSpec((R, DE), lambda i: (i, 0)),
            pl.BlockSpec((R, 1), lambda i: (i, 0)),
            pl.BlockSpec((DE, F), lambda i: (0, 0)),
            pl.BlockSpec((1, F), lambda i: (0, 0)),
            pl.BlockSpec((1, F), lambda i: (0, 0)),
            pl.BlockSpec((1, F), lambda i: (0, 0)),
            smem((1, 1), lambda i: (0, 0)),
            smem((1, 1), lambda i: (0, 0)),
        ],
        out_specs=[
            pl.BlockSpec((R, F), lambda i: (i, 0)),
            pl.BlockSpec((R * F,), lambda i: (i,)),
        ],
        out_shape=[
            jax.ShapeDtypeStruct((BE, F), jnp.float32),
            jax.ShapeDtypeStruct((BE * F,), jnp.float32),
        ],
    )(t, ef, dist, w3, bm, gm, bem, sig, beta)


def _qkvg_body(n_ref, at_ref, wt_ref, wb_ref, b_ref, oq_ref, ok_ref, ov_ref,
               og_ref):
    F = oq_ref.shape[-1]
    y = jnp.dot(n_ref[0], wt_ref[...], preferred_element_type=jnp.float32)
    y += lax.dot_general(at_ref[...], wb_ref[...], (((0,), (0,)), ((), ())),
                         preferred_element_type=jnp.float32)
    y += b_ref[...]
    oq_ref[0] = y[:, :F]
    ok_ref[0] = y[:, F:2 * F]
    ov_ref[0] = y[:, 2 * F:3 * F]
    og_ref[0] = jax.nn.sigmoid(y[:, 3 * F:])


def _qkvg(nodes, agg_t, wtop, wbot, bcat):
    B, N, F = nodes.shape
    RP = 512
    grid = (B, N // RP)
    out = jax.ShapeDtypeStruct((B, N, F), jnp.float32)
    nrp = N // RP
    return pl.pallas_call(
        _qkvg_body,
        grid=grid,
        in_specs=[
            pl.BlockSpec((1, RP, F), lambda b, i: (b, i, 0)),
            pl.BlockSpec((F, RP), lambda b, i: (0, b * nrp + i)),
            pl.BlockSpec((F, 4 * F), lambda b, i: (0, 0)),
            pl.BlockSpec((F, 4 * F), lambda b, i: (0, 0)),
            pl.BlockSpec((1, 4 * F), lambda b, i: (0, 0)),
        ],
        out_specs=[
            pl.BlockSpec((1, RP, F), lambda b, i: (b, i, 0)),
            pl.BlockSpec((1, RP, F), lambda b, i: (b, i, 0)),
            pl.BlockSpec((1, RP, F), lambda b, i: (b, i, 0)),
            pl.BlockSpec((1, RP, F), lambda b, i: (b, i, 0)),
        ],
        out_shape=[out, out, out, out],
    )(nodes, agg_t, wtop, wbot, bcat)


def _attn_body(q_ref, k_ref, v_ref, g_ref, wo_ref, bo_ref, gu_ref, bu_ref,
               o_ref, *, H):
    q = q_ref[0]
    k = k_ref[0]
    v = v_ref[0]
    gt = g_ref[0]
    F = q.shape[-1]
    dh = F // H
    scale = 1.0 / math.sqrt(dh)
    outs = []
    for h in range(H):
        qh = q[:, h * dh:(h + 1) * dh]
        kh = k[:, h * dh:(h + 1) * dh]
        vh = v[:, h * dh:(h + 1) * dh]
        s = lax.dot_general(qh, kh, (((1,), (1,)), ((), ())),
                            preferred_element_type=jnp.float32) * scale
        mx = jnp.max(s, axis=-1, keepdims=True)
        p = jnp.exp(s - mx)
        wgt = p / jnp.sum(p, axis=-1, keepdims=True)
        outs.append(jnp.dot(wgt, vh, preferred_element_type=jnp.float32))
    att = jnp.concatenate(outs, axis=1) * gt
    out = jnp.dot(att, wo_ref[...], preferred_element_type=jnp.float32) + bo_ref[...]
    out = _gelu(out)
    mu = jnp.mean(out, axis=-1, keepdims=True)
    xc = out - mu
    var = jnp.mean(xc * xc, axis=-1, keepdims=True)
    o_ref[0] = xc * lax.rsqrt(var + 1e-3) * gu_ref[...] + bu_ref[...]


def _attention(q, k, v, gt, wo, bo, gu, bu, H):
    B, N, F = q.shape
    QB = 256
    grid = (B, N // QB)
    return pl.pallas_call(
        functools.partial(_attn_body, H=H),
        grid=grid,
        in_specs=[
            pl.BlockSpec((1, QB, F), lambda b, i: (b, i, 0)),
            pl.BlockSpec((1, N, F), lambda b, i: (b, 0, 0)),
            pl.BlockSpec((1, N, F), lambda b, i: (b, 0, 0)),
            pl.BlockSpec((1, QB, F), lambda b, i: (b, i, 0)),
            pl.BlockSpec((F, F), lambda b, i: (0, 0)),
            pl.BlockSpec((1, F), lambda b, i: (0, 0)),
            pl.BlockSpec((1, F), lambda b, i: (0, 0)),
            pl.BlockSpec((1, F), lambda b, i: (0, 0)),
        ],
        out_specs=pl.BlockSpec((1, QB, F), lambda b, i: (b, i, 0)),
        out_shape=jax.ShapeDtypeStruct((B, N, F), jnp.float32),
    )(q, k, v, gt, wo, bo, gu, bu)


def kernel(nodes, edge_features, distance, edges, W_msg, b_msg, g_msg, be_msg,
           Wq, bq, Wk, bk, Wv, bv, Wg, bg, Wo, bo, g_upd, be_upd, sigma,
           beta_p):
    B, N, F = nodes.shape
    E = edges.shape[1]
    DE = edge_features.shape[-1]
    H = 8
    BE = B * E

    w1 = W_msg[:F]
    w2 = W_msg[F:2 * F]
    w3 = W_msg[2 * F:]

    p1, p2 = _proj(nodes, w1, w2)

    e0 = edges[..., 0].astype(jnp.int32)
    e1 = edges[..., 1].astype(jnp.int32)
    boff = (jnp.arange(B, dtype=jnp.int32) * N)[:, None]
    idx0 = (e0 + boff).reshape(BE)
    idx1 = (e1 + boff).reshape(BE)
    t = _make_gather(BE, F)(p1.reshape(B * N, F), p2.reshape(B * N, F),
                            idx0, idx1)

    wm, wm_t = _edge_stage(t, edge_features.reshape(BE, DE),
                     distance.reshape(BE, 1), w3,
                     b_msg.reshape(1, F), g_msg.reshape(1, F),
                     be_msg.reshape(1, F), sigma.reshape(1, 1),
                     beta_p.reshape(1, 1))

    bpc = B // NC
    locoff = ((jnp.arange(B, dtype=jnp.int32) % bpc) * N)[:, None]
    idx_sc = (e1 + locoff).reshape(BE)
    lane_off = jnp.arange(LANES, dtype=jnp.int32) * (bpc * N)
    pos1d = (idx_sc[:, None] + lane_off[None, :]).reshape(BE * LANES)
    agg1d = _make_scatter(B, E, N, F)(wm_t, pos1d)
    agg_t = agg1d.reshape(NC, NS, LANES, bpc * N).transpose(1, 2, 0, 3)
    agg_t = agg_t.reshape(F, B * N)

    wcat = jnp.concatenate([Wq, Wk, Wv, Wg], axis=1)
    bcat = jnp.concatenate([bq, bk, bv, bg]).reshape(1, 4 * F)
    q, k, v, gt = _qkvg(nodes, agg_t, wcat[:F], wcat[F:], bcat)
    updated = _attention(q, k, v, gt, Wo, bo.reshape(1, F),
                         g_upd.reshape(1, F), be_upd.reshape(1, F), H)

    return (updated, wm.reshape(B, E, F), distance, edges)


# speedup vs baseline: 6.0192x; 1.6887x over previous
import functools
import math

import jax
import jax.numpy as jnp
from jax import lax
from jax.experimental import pallas as pl
from jax.experimental.pallas import tpu as pltpu
from jax.experimental.pallas import tpu_sc as plsc

NC = 2
NS = 16
LANES = 16


def _proj_body(x_ref, w1_ref, w2_ref, o1_ref, o2_ref):
    x = x_ref[0]
    o1_ref[0] = jnp.dot(x, w1_ref[...], preferred_element_type=jnp.float32)
    o2_ref[0] = jnp.dot(x, w2_ref[...], preferred_element_type=jnp.float32)


def _proj(nodes, w1, w2):
    B, N, F = nodes.shape
    RP = 512
    grid = (B, N // RP)
    out = jax.ShapeDtypeStruct((B, N, F), jnp.float32)
    return pl.pallas_call(
        _proj_body,
        grid=grid,
        in_specs=[
            pl.BlockSpec((1, RP, F), lambda b, i: (b, i, 0)),
            pl.BlockSpec((F, F), lambda b, i: (0, 0)),
            pl.BlockSpec((F, F), lambda b, i: (0, 0)),
        ],
        out_specs=[
            pl.BlockSpec((1, RP, F), lambda b, i: (b, i, 0)),
            pl.BlockSpec((1, RP, F), lambda b, i: (b, i, 0)),
        ],
        out_shape=[out, out],
    )(nodes, w1, w2)


def _make_gather(BE, F, C=128):
    n_w = NC * NS
    per_w = BE // n_w
    n_chunks = per_w // C
    mesh = plsc.VectorSubcoreMesh(core_axis_name="c", subcore_axis_name="s")

    def body(p1, p2, i0, i1, t, i0_v, i1_v, r0_v, r1_v, sem0, sem1):
        wid = lax.axis_index("s") * NC + lax.axis_index("c")
        base = wid * per_w

        def chunk(i, carry):
            off = base + i * C
            pltpu.sync_copy(i0.at[pl.ds(off, C)], i0_v)
            pltpu.sync_copy(i1.at[pl.ds(off, C)], i1_v)
            cp0 = pltpu.async_copy(p1.at[i0_v], r0_v, sem0)
            cp1 = pltpu.async_copy(p2.at[i1_v], r1_v, sem1)
            cp0.wait()
            cp1.wait()

            def add_row(r, c2):
                for j in range(F // LANES):
                    sl = pl.ds(j * LANES, LANES)
                    r0_v[r, sl] = r0_v[r, sl] + r1_v[r, sl]
                return c2

            lax.fori_loop(0, C, add_row, 0)
            pltpu.sync_copy(r0_v, t.at[pl.ds(off, C)])
            return carry

        lax.fori_loop(0, n_chunks, chunk, 0)

    return pl.kernel(
        body,
        mesh=mesh,
        compiler_params=pltpu.CompilerParams(needs_layout_passes=False),
        out_type=jax.ShapeDtypeStruct((BE, F), jnp.float32),
        scratch_types=[
            pltpu.VMEM((C,), jnp.int32),
            pltpu.VMEM((C,), jnp.int32),
            pltpu.VMEM((C, F), jnp.float32),
            pltpu.VMEM((C, F), jnp.float32),
            pltpu.SemaphoreType.DMA,
            pltpu.SemaphoreType.DMA,
        ],
    )


def _make_scatter(B, E, N, F, C=256):
    bpc = B // NC
    rows = bpc * N
    n_edges = bpc * E
    n_chunks = n_edges // C
    blk = LANES * C
    stride = rows + 1
    acc_sz = LANES * stride
    out_sz = LANES * rows
    assert F == NS * LANES
    mesh = plsc.VectorSubcoreMesh(core_axis_name="c", subcore_axis_name="s")

    tstride = LANES + 1

    def body(wm1d, pos1d, agg1d, pos_v0, pos_v1, wm_v0, wm_v1, wt_v, acc_v,
             sem0, sem1):
        c = lax.axis_index("c")
        s = lax.axis_index("s")
        iota = lax.iota(jnp.int32, LANES)
        zeros = jnp.zeros((LANES,), jnp.float32)
        pos_bufs = (pos_v0, pos_v1)
        wm_bufs = (wm_v0, wm_v1)
        sems = (sem0, sem1)

        @plsc.parallel_loop(0, acc_sz // LANES, unroll=8)
        def _zrow(i):
            acc_v[pl.ds(i * LANES, LANES)] = zeros

        def issue(g, b):
            pltpu.async_copy(pos1d.at[pl.ds(g * C * LANES, C * LANES)],
                             pos_bufs[b], sems[b])
            pltpu.async_copy(wm1d.at[pl.ds((g * NS + s) * blk, blk)],
                             wm_bufs[b], sems[b])

        def drain(g, b):
            pltpu.make_async_copy(pos1d.at[pl.ds(g * C * LANES, C * LANES)],
                                  pos_bufs[b], sems[b]).wait()
            pltpu.make_async_copy(wm1d.at[pl.ds((g * NS + s) * blk, blk)],
                                  wm_bufs[b], sems[b]).wait()

        g0 = c * n_chunks
        issue(g0, 0)

        def outer(i, carry):
            for b in range(2):
                li = 2 * i + b
                g = g0 + li
                drain(g, b)

                @pl.when(li + 1 < n_chunks)
                def _():
                    issue(g + 1, b ^ 1)

                pv = pos_bufs[b]
                wv = wm_bufs[b]

                @plsc.parallel_loop(0, C * LANES // LANES, unroll=8)
                def _tr(i):
                    l = i // (C // LANES)
                    eg = i % (C // LANES)
                    vals = wv[pl.ds(l * C + eg * LANES, LANES)]
                    plsc.store_scatter(
                        wt_v, [iota * tstride + (eg * LANES * tstride + l)],
                        vals)

                @plsc.parallel_loop(0, C, unroll=8)
                def _edge(e):
                    pos = pv[pl.ds(e * LANES, LANES)]
                    val = plsc.load_gather(wt_v, [iota + e * tstride])
                    plsc.addupdate_scatter(acc_v, [pos], val)
            return carry

        lax.fori_loop(0, n_chunks // 2, outer, 0)
        base = (c * NS + s) * out_sz
        for l in range(LANES):
            lb = l * stride

            @plsc.parallel_loop(0, rows // LANES, unroll=8)
            def _cp(i):
                wm_v0[pl.ds(i * LANES, LANES)] = plsc.load_gather(
                    acc_v, [iota + (lb + i * LANES)])

            pltpu.sync_copy(wm_v0, agg1d.at[pl.ds(base + l * rows, rows)])

    return pl.kernel(
        body,
        mesh=mesh,
        compiler_params=pltpu.CompilerParams(needs_layout_passes=False),
        out_type=jax.ShapeDtypeStruct((NC * NS * out_sz,), jnp.float32),
        scratch_types=[
            pltpu.VMEM((C * LANES,), jnp.int32),
            pltpu.VMEM((C * LANES,), jnp.int32),
            pltpu.VMEM((blk,), jnp.float32),
            pltpu.VMEM((blk,), jnp.float32),
            pltpu.VMEM((C * (LANES + 1),), jnp.float32),
            pltpu.VMEM((acc_sz,), jnp.float32),
            pltpu.SemaphoreType.DMA,
            pltpu.SemaphoreType.DMA,
        ],
    )


def _gelu(x):
    return 0.5 * x * (1.0 + lax.erf(x * (1.0 / math.sqrt(2.0))))


def _edge_body(t_ref, ef_ref, d_ref, w3_ref, bm_ref, g_ref, be_ref, sig_ref,
               beta_ref, o_ref, ot_ref):
    u = t_ref[...] + jnp.dot(ef_ref[...], w3_ref[...],
                             preferred_element_type=jnp.float32) + bm_ref[...]
    g = _gelu(u)
    mu = jnp.mean(g, axis=-1, keepdims=True)
    xc = g - mu
    var = jnp.mean(xc * xc, axis=-1, keepdims=True)
    m = xc * lax.rsqrt(var + 1e-3) * g_ref[...] + be_ref[...]
    sig = sig_ref[0, 0]
    beta = beta_ref[0, 0]
    d = d_ref[...]
    xw = (d * d) * (0.5 / (sig * sig))
    w = jnp.exp(-jnp.exp(beta * jnp.log(xw)))
    wm = m * w
    o_ref[...] = wm
    R, F = wm.shape
    CC = 256
    for cc in range(R // CC):
        mcc = wm[cc * CC:(cc + 1) * CC, :].T
        ot_ref[pl.ds(cc * CC * F, CC * F)] = mcc.reshape(CC * F)


def _edge_stage(t, ef, dist, w3, bm, gm, bem, sig, beta):
    BE, F = t.shape
    DE = ef.shape[-1]
    R = 1024
    grid = (BE // R,)
    smem = functools.partial(pl.BlockSpec, memory_space=pltpu.SMEM)
    return pl.pallas_call(
        _edge_body,
        grid=grid,
        in_specs=[
            pl.BlockSpec((R, F), lambda i: (i, 0)),
            pl.BlockSpec((R, DE), lambda i: (i, 0)),
            pl.BlockSpec((R, 1), lambda i: (i, 0)),
            pl.BlockSpec((DE, F), lambda i: (0, 0)),
            pl.BlockSpec((1, F), lambda i: (0, 0)),
            pl.BlockSpec((1, F), lambda i: (0, 0)),
            pl.BlockSpec((1, F), lambda i: (0, 0)),
            smem((1, 1), lambda i: (0, 0)),
            smem((1, 1), lambda i: (0, 0)),
        ],
        out_specs=[
            pl.BlockSpec((R, F), lambda i: (i, 0)),
            pl.BlockSpec((R * F,), lambda i: (i,)),
        ],
        out_shape=[
            jax.ShapeDtypeStruct((BE, F), jnp.float32),
            jax.ShapeDtypeStruct((BE * F,), jnp.float32),
        ],
    )(t, ef, dist, w3, bm, gm, bem, sig, beta)


def _qkvg_body(n_ref, at_ref, wt_ref, wb_ref, b_ref, oq_ref, ok_ref, ov_ref,
               og_ref):
    F = oq_ref.shape[-1]
    y = jnp.dot(n_ref[0], wt_ref[...], preferred_element_type=jnp.float32)
    y += lax.dot_general(at_ref[...], wb_ref[...], (((0,), (0,)), ((), ())),
                         preferred_element_type=jnp.float32)
    y += b_ref[...]
    oq_ref[0] = y[:, :F]
    ok_ref[0] = y[:, F:2 * F]
    ov_ref[0] = y[:, 2 * F:3 * F]
    og_ref[0] = jax.nn.sigmoid(y[:, 3 * F:])


def _qkvg(nodes, agg_t, wtop, wbot, bcat):
    B, N, F = nodes.shape
    RP = 512
    grid = (B, N // RP)
    out = jax.ShapeDtypeStruct((B, N, F), jnp.float32)
    nrp = N // RP
    return pl.pallas_call(
        _qkvg_body,
        grid=grid,
        in_specs=[
            pl.BlockSpec((1, RP, F), lambda b, i: (b, i, 0)),
            pl.BlockSpec((F, RP), lambda b, i: (0, b * nrp + i)),
            pl.BlockSpec((F, 4 * F), lambda b, i: (0, 0)),
            pl.BlockSpec((F, 4 * F), lambda b, i: (0, 0)),
            pl.BlockSpec((1, 4 * F), lambda b, i: (0, 0)),
        ],
        out_specs=[
            pl.BlockSpec((1, RP, F), lambda b, i: (b, i, 0)),
            pl.BlockSpec((1, RP, F), lambda b, i: (b, i, 0)),
            pl.BlockSpec((1, RP, F), lambda b, i: (b, i, 0)),
            pl.BlockSpec((1, RP, F), lambda b, i: (b, i, 0)),
        ],
        out_shape=[out, out, out, out],
    )(nodes, agg_t, wtop, wbot, bcat)


def _attn_body(q_ref, k_ref, v_ref, g_ref, wo_ref, bo_ref, gu_ref, bu_ref,
               o_ref, *, H):
    q = q_ref[0]
    k = k_ref[0]
    v = v_ref[0]
    gt = g_ref[0]
    F = q.shape[-1]
    dh = F // H
    scale = 1.0 / math.sqrt(dh)
    outs = []
    for h in range(H):
        qh = q[:, h * dh:(h + 1) * dh]
        kh = k[:, h * dh:(h + 1) * dh]
        vh = v[:, h * dh:(h + 1) * dh]
        s = lax.dot_general(qh, kh, (((1,), (1,)), ((), ())),
                            preferred_element_type=jnp.float32) * scale
        mx = jnp.max(s, axis=-1, keepdims=True)
        p = jnp.exp(s - mx)
        wgt = p / jnp.sum(p, axis=-1, keepdims=True)
        outs.append(jnp.dot(wgt, vh, preferred_element_type=jnp.float32))
    att = jnp.concatenate(outs, axis=1) * gt
    out = jnp.dot(att, wo_ref[...], preferred_element_type=jnp.float32) + bo_ref[...]
    out = _gelu(out)
    mu = jnp.mean(out, axis=-1, keepdims=True)
    xc = out - mu
    var = jnp.mean(xc * xc, axis=-1, keepdims=True)
    o_ref[0] = xc * lax.rsqrt(var + 1e-3) * gu_ref[...] + bu_ref[...]


def _attention(q, k, v, gt, wo, bo, gu, bu, H):
    B, N, F = q.shape
    QB = 256
    grid = (B, N // QB)
    return pl.pallas_call(
        functools.partial(_attn_body, H=H),
        grid=grid,
        in_specs=[
            pl.BlockSpec((1, QB, F), lambda b, i: (b, i, 0)),
            pl.BlockSpec((1, N, F), lambda b, i: (b, 0, 0)),
            pl.BlockSpec((1, N, F), lambda b, i: (b, 0, 0)),
            pl.BlockSpec((1, QB, F), lambda b, i: (b, i, 0)),
            pl.BlockSpec((F, F), lambda b, i: (0, 0)),
            pl.BlockSpec((1, F), lambda b, i: (0, 0)),
            pl.BlockSpec((1, F), lambda b, i: (0, 0)),
            pl.BlockSpec((1, F), lambda b, i: (0, 0)),
        ],
        out_specs=pl.BlockSpec((1, QB, F), lambda b, i: (b, i, 0)),
        out_shape=jax.ShapeDtypeStruct((B, N, F), jnp.float32),
    )(q, k, v, gt, wo, bo, gu, bu)


def kernel(nodes, edge_features, distance, edges, W_msg, b_msg, g_msg, be_msg,
           Wq, bq, Wk, bk, Wv, bv, Wg, bg, Wo, bo, g_upd, be_upd, sigma,
           beta_p):
    B, N, F = nodes.shape
    E = edges.shape[1]
    DE = edge_features.shape[-1]
    H = 8
    BE = B * E

    w1 = W_msg[:F]
    w2 = W_msg[F:2 * F]
    w3 = W_msg[2 * F:]

    p1, p2 = _proj(nodes, w1, w2)

    e0 = edges[..., 0].astype(jnp.int32)
    e1 = edges[..., 1].astype(jnp.int32)
    boff = (jnp.arange(B, dtype=jnp.int32) * N)[:, None]
    idx0 = (e0 + boff).reshape(BE)
    idx1 = (e1 + boff).reshape(BE)
    t = _make_gather(BE, F)(p1.reshape(B * N, F), p2.reshape(B * N, F),
                            idx0, idx1)

    wm, wm_t = _edge_stage(t, edge_features.reshape(BE, DE),
                     distance.reshape(BE, 1), w3,
                     b_msg.reshape(1, F), g_msg.reshape(1, F),
                     be_msg.reshape(1, F), sigma.reshape(1, 1),
                     beta_p.reshape(1, 1))

    bpc = B // NC
    locoff = ((jnp.arange(B, dtype=jnp.int32) % bpc) * N)[:, None]
    idx_sc = (e1 + locoff).reshape(BE)
    lane_off = jnp.arange(LANES, dtype=jnp.int32) * (bpc * N + 1)
    pos1d = (idx_sc[:, None] + lane_off[None, :]).reshape(BE * LANES)
    agg1d = _make_scatter(B, E, N, F)(wm_t, pos1d)
    agg_t = agg1d.reshape(NC, NS, LANES, bpc * N).transpose(1, 2, 0, 3)
    agg_t = agg_t.reshape(F, B * N)

    wcat = jnp.concatenate([Wq, Wk, Wv, Wg], axis=1)
    bcat = jnp.concatenate([bq, bk, bv, bg]).reshape(1, 4 * F)
    q, k, v, gt = _qkvg(nodes, agg_t, wcat[:F], wcat[F:], bcat)
    updated = _attention(q, k, v, gt, Wo, bo.reshape(1, F),
                         g_upd.reshape(1, F), be_upd.reshape(1, F), H)

    return (updated, wm.reshape(B, E, F), distance, edges)


# speedup vs baseline: 6.0214x; 1.0004x over previous
import functools
import math

import jax
import jax.numpy as jnp
from jax import lax
from jax.experimental import pallas as pl
from jax.experimental.pallas import tpu as pltpu
from jax.experimental.pallas import tpu_sc as plsc

NC = 2
NS = 16
LANES = 16


def _proj_body(x_ref, w1_ref, w2_ref, o1_ref, o2_ref):
    x = x_ref[0]
    o1_ref[0] = jnp.dot(x, w1_ref[...], preferred_element_type=jnp.float32)
    o2_ref[0] = jnp.dot(x, w2_ref[...], preferred_element_type=jnp.float32)


def _proj(nodes, w1, w2):
    B, N, F = nodes.shape
    RP = 512
    grid = (B, N // RP)
    out = jax.ShapeDtypeStruct((B, N, F), jnp.float32)
    return pl.pallas_call(
        _proj_body,
        grid=grid,
        in_specs=[
            pl.BlockSpec((1, RP, F), lambda b, i: (b, i, 0)),
            pl.BlockSpec((F, F), lambda b, i: (0, 0)),
            pl.BlockSpec((F, F), lambda b, i: (0, 0)),
        ],
        out_specs=[
            pl.BlockSpec((1, RP, F), lambda b, i: (b, i, 0)),
            pl.BlockSpec((1, RP, F), lambda b, i: (b, i, 0)),
        ],
        out_shape=[out, out],
    )(nodes, w1, w2)


def _make_gather(BE, F, C=128):
    n_w = NC * NS
    per_w = BE // n_w
    n_chunks = per_w // C
    mesh = plsc.VectorSubcoreMesh(core_axis_name="c", subcore_axis_name="s")

    def body(p1, p2, i0, i1, t, i0_v, i1_v, r0_v, r1_v, sem0, sem1):
        wid = lax.axis_index("s") * NC + lax.axis_index("c")
        base = wid * per_w

        def chunk(i, carry):
            off = base + i * C
            pltpu.sync_copy(i0.at[pl.ds(off, C)], i0_v)
            pltpu.sync_copy(i1.at[pl.ds(off, C)], i1_v)
            cp0 = pltpu.async_copy(p1.at[i0_v], r0_v, sem0)
            cp1 = pltpu.async_copy(p2.at[i1_v], r1_v, sem1)
            cp0.wait()
            cp1.wait()

            @plsc.parallel_loop(0, C * (F // LANES), unroll=8)
            def _add(k):
                r = k // (F // LANES)
                sl = pl.ds((k % (F // LANES)) * LANES, LANES)
                r0_v[r, sl] = r0_v[r, sl] + r1_v[r, sl]

            pltpu.sync_copy(r0_v, t.at[pl.ds(off, C)])
            return carry

        lax.fori_loop(0, n_chunks, chunk, 0)

    return pl.kernel(
        body,
        mesh=mesh,
        compiler_params=pltpu.CompilerParams(needs_layout_passes=False),
        out_type=jax.ShapeDtypeStruct((BE, F), jnp.float32),
        scratch_types=[
            pltpu.VMEM((C,), jnp.int32),
            pltpu.VMEM((C,), jnp.int32),
            pltpu.VMEM((C, F), jnp.float32),
            pltpu.VMEM((C, F), jnp.float32),
            pltpu.SemaphoreType.DMA,
            pltpu.SemaphoreType.DMA,
        ],
    )


def _make_scatter(B, E, N, F, C=256):
    bpc = B // NC
    rows = bpc * N
    n_edges = bpc * E
    n_chunks = n_edges // C
    blk = LANES * C
    stride = rows + 1
    acc_sz = LANES * stride
    out_sz = LANES * rows
    assert F == NS * LANES
    mesh = plsc.VectorSubcoreMesh(core_axis_name="c", subcore_axis_name="s")

    tstride = LANES + 1

    def body(wm1d, pos1d, agg1d, pos_v0, pos_v1, wm_v0, wm_v1, wt_v, acc_v,
             sem0, sem1):
        c = lax.axis_index("c")
        s = lax.axis_index("s")
        iota = lax.iota(jnp.int32, LANES)
        zeros = jnp.zeros((LANES,), jnp.float32)
        pos_bufs = (pos_v0, pos_v1)
        wm_bufs = (wm_v0, wm_v1)
        sems = (sem0, sem1)

        @plsc.parallel_loop(0, acc_sz // LANES, unroll=8)
        def _zrow(i):
            acc_v[pl.ds(i * LANES, LANES)] = zeros

        def issue(g, b):
            pltpu.async_copy(pos1d.at[pl.ds(g * C * LANES, C * LANES)],
                             pos_bufs[b], sems[b])
            pltpu.async_copy(wm1d.at[pl.ds((g * NS + s) * blk, blk)],
                             wm_bufs[b], sems[b])

        def drain(g, b):
            pltpu.make_async_copy(pos1d.at[pl.ds(g * C * LANES, C * LANES)],
                                  pos_bufs[b], sems[b]).wait()
            pltpu.make_async_copy(wm1d.at[pl.ds((g * NS + s) * blk, blk)],
                                  wm_bufs[b], sems[b]).wait()

        g0 = c * n_chunks
        issue(g0, 0)

        def outer(i, carry):
            for b in range(2):
                li = 2 * i + b
                g = g0 + li
                drain(g, b)

                @pl.when(li + 1 < n_chunks)
                def _():
                    issue(g + 1, b ^ 1)

                pv = pos_bufs[b]
                wv = wm_bufs[b]

                @plsc.parallel_loop(0, C * LANES // LANES, unroll=8)
                def _tr(i):
                    l = i // (C // LANES)
                    eg = i % (C // LANES)
                    vals = wv[pl.ds(l * C + eg * LANES, LANES)]
                    plsc.store_scatter(
                        wt_v, [iota * tstride + (eg * LANES * tstride + l)],
                        vals)

                @plsc.parallel_loop(0, C, unroll=8)
                def _edge(e):
                    pos = pv[pl.ds(e * LANES, LANES)]
                    val = plsc.load_gather(wt_v, [iota + e * tstride])
                    plsc.addupdate_scatter(acc_v, [pos], val)
            return carry

        lax.fori_loop(0, n_chunks // 2, outer, 0)
        base = (c * NS + s) * out_sz
        for l in range(LANES):
            lb = l * stride

            @plsc.parallel_loop(0, rows // LANES, unroll=8)
            def _cp(i):
                wm_v0[pl.ds(i * LANES, LANES)] = plsc.load_gather(
                    acc_v, [iota + (lb + i * LANES)])

            pltpu.sync_copy(wm_v0, agg1d.at[pl.ds(base + l * rows, rows)])

    return pl.kernel(
        body,
        mesh=mesh,
        compiler_params=pltpu.CompilerParams(needs_layout_passes=False),
        out_type=jax.ShapeDtypeStruct((NC * NS * out_sz,), jnp.float32),
        scratch_types=[
            pltpu.VMEM((C * LANES,), jnp.int32),
            pltpu.VMEM((C * LANES,), jnp.int32),
            pltpu.VMEM((blk,), jnp.float32),
            pltpu.VMEM((blk,), jnp.float32),
            pltpu.VMEM((C * (LANES + 1),), jnp.float32),
            pltpu.VMEM((acc_sz,), jnp.float32),
            pltpu.SemaphoreType.DMA,
            pltpu.SemaphoreType.DMA,
        ],
    )


def _gelu(x):
    return 0.5 * x * (1.0 + lax.erf(x * (1.0 / math.sqrt(2.0))))


def _edge_body(t_ref, ef_ref, d_ref, w3_ref, bm_ref, g_ref, be_ref, sig_ref,
               beta_ref, o_ref, ot_ref):
    u = t_ref[...] + jnp.dot(ef_ref[...], w3_ref[...],
                             preferred_element_type=jnp.float32) + bm_ref[...]
    g = _gelu(u)
    mu = jnp.mean(g, axis=-1, keepdims=True)
    xc = g - mu
    var = jnp.mean(xc * xc, axis=-1, keepdims=True)
    m = xc * lax.rsqrt(var + 1e-3) * g_ref[...] + be_ref[...]
    sig = sig_ref[0, 0]
    beta = beta_ref[0, 0]
    d = d_ref[...]
    xw = (d * d) * (0.5 / (sig * sig))
    w = jnp.exp(-jnp.exp(beta * jnp.log(xw)))
    wm = m * w
    o_ref[...] = wm
    R, F = wm.shape
    CC = 256
    for cc in range(R // CC):
        mcc = wm[cc * CC:(cc + 1) * CC, :].T
        ot_ref[pl.ds(cc * CC * F, CC * F)] = mcc.reshape(CC * F)


def _edge_stage(t, ef, dist, w3, bm, gm, bem, sig, beta):
    BE, F = t.shape
    DE = ef.shape[-1]
    R = 1024
    grid = (BE // R,)
    smem = functools.partial(pl.BlockSpec, memory_space=pltpu.SMEM)
    return pl.pallas_call(
        _edge_body,
        grid=grid,
        in_specs=[
            pl.BlockSpec((R, F), lambda i: (i, 0)),
            pl.BlockSpec((R, DE), lambda i: (i, 0)),
            pl.BlockSpec((R, 1), lambda i: (i, 0)),
            pl.BlockSpec((DE, F), lambda i: (0, 0)),
            pl.BlockSpec((1, F), lambda i: (0, 0)),
            pl.BlockSpec((1, F), lambda i: (0, 0)),
            pl.BlockSpec((1, F), lambda i: (0, 0)),
            smem((1, 1), lambda i: (0, 0)),
            smem((1, 1), lambda i: (0, 0)),
        ],
        out_specs=[
            pl.BlockSpec((R, F), lambda i: (i, 0)),
            pl.BlockSpec((R * F,), lambda i: (i,)),
        ],
        out_shape=[
            jax.ShapeDtypeStruct((BE, F), jnp.float32),
            jax.ShapeDtypeStruct((BE * F,), jnp.float32),
        ],
    )(t, ef, dist, w3, bm, gm, bem, sig, beta)


def _qkvg_body(n_ref, at_ref, wt_ref, wb_ref, b_ref, oq_ref, ok_ref, ov_ref,
               og_ref):
    F = oq_ref.shape[-1]
    y = jnp.dot(n_ref[0], wt_ref[...], preferred_element_type=jnp.float32)
    y += lax.dot_general(at_ref[...], wb_ref[...], (((0,), (0,)), ((), ())),
                         preferred_element_type=jnp.float32)
    y += b_ref[...]
    oq_ref[0] = y[:, :F]
    ok_ref[0] = y[:, F:2 * F]
    ov_ref[0] = y[:, 2 * F:3 * F]
    og_ref[0] = jax.nn.sigmoid(y[:, 3 * F:])


def _qkvg(nodes, agg_t, wtop, wbot, bcat):
    B, N, F = nodes.shape
    RP = 512
    grid = (B, N // RP)
    out = jax.ShapeDtypeStruct((B, N, F), jnp.float32)
    nrp = N // RP
    return pl.pallas_call(
        _qkvg_body,
        grid=grid,
        in_specs=[
            pl.BlockSpec((1, RP, F), lambda b, i: (b, i, 0)),
            pl.BlockSpec((F, RP), lambda b, i: (0, b * nrp + i)),
            pl.BlockSpec((F, 4 * F), lambda b, i: (0, 0)),
            pl.BlockSpec((F, 4 * F), lambda b, i: (0, 0)),
            pl.BlockSpec((1, 4 * F), lambda b, i: (0, 0)),
        ],
        out_specs=[
            pl.BlockSpec((1, RP, F), lambda b, i: (b, i, 0)),
            pl.BlockSpec((1, RP, F), lambda b, i: (b, i, 0)),
            pl.BlockSpec((1, RP, F), lambda b, i: (b, i, 0)),
            pl.BlockSpec((1, RP, F), lambda b, i: (b, i, 0)),
        ],
        out_shape=[out, out, out, out],
    )(nodes, agg_t, wtop, wbot, bcat)


def _attn_body(q_ref, k_ref, v_ref, g_ref, wo_ref, bo_ref, gu_ref, bu_ref,
               o_ref, *, H):
    q = q_ref[0]
    k = k_ref[0]
    v = v_ref[0]
    gt = g_ref[0]
    F = q.shape[-1]
    dh = F // H
    scale = 1.0 / math.sqrt(dh)
    outs = []
    for h in range(H):
        qh = q[:, h * dh:(h + 1) * dh]
        kh = k[:, h * dh:(h + 1) * dh]
        vh = v[:, h * dh:(h + 1) * dh]
        s = lax.dot_general(qh, kh, (((1,), (1,)), ((), ())),
                            preferred_element_type=jnp.float32) * scale
        mx = jnp.max(s, axis=-1, keepdims=True)
        p = jnp.exp(s - mx)
        wgt = p / jnp.sum(p, axis=-1, keepdims=True)
        outs.append(jnp.dot(wgt, vh, preferred_element_type=jnp.float32))
    att = jnp.concatenate(outs, axis=1) * gt
    out = jnp.dot(att, wo_ref[...], preferred_element_type=jnp.float32) + bo_ref[...]
    out = _gelu(out)
    mu = jnp.mean(out, axis=-1, keepdims=True)
    xc = out - mu
    var = jnp.mean(xc * xc, axis=-1, keepdims=True)
    o_ref[0] = xc * lax.rsqrt(var + 1e-3) * gu_ref[...] + bu_ref[...]


def _attention(q, k, v, gt, wo, bo, gu, bu, H):
    B, N, F = q.shape
    QB = 256
    grid = (B, N // QB)
    return pl.pallas_call(
        functools.partial(_attn_body, H=H),
        grid=grid,
        in_specs=[
            pl.BlockSpec((1, QB, F), lambda b, i: (b, i, 0)),
            pl.BlockSpec((1, N, F), lambda b, i: (b, 0, 0)),
            pl.BlockSpec((1, N, F), lambda b, i: (b, 0, 0)),
            pl.BlockSpec((1, QB, F), lambda b, i: (b, i, 0)),
            pl.BlockSpec((F, F), lambda b, i: (0, 0)),
            pl.BlockSpec((1, F), lambda b, i: (0, 0)),
            pl.BlockSpec((1, F), lambda b, i: (0, 0)),
            pl.BlockSpec((1, F), lambda b, i: (0, 0)),
        ],
        out_specs=pl.BlockSpec((1, QB, F), lambda b, i: (b, i, 0)),
        out_shape=jax.ShapeDtypeStruct((B, N, F), jnp.float32),
    )(q, k, v, gt, wo, bo, gu, bu)


def kernel(nodes, edge_features, distance, edges, W_msg, b_msg, g_msg, be_msg,
           Wq, bq, Wk, bk, Wv, bv, Wg, bg, Wo, bo, g_upd, be_upd, sigma,
           beta_p):
    B, N, F = nodes.shape
    E = edges.shape[1]
    DE = edge_features.shape[-1]
    H = 8
    BE = B * E

    w1 = W_msg[:F]
    w2 = W_msg[F:2 * F]
    w3 = W_msg[2 * F:]

    p1, p2 = _proj(nodes, w1, w2)

    e0 = edges[..., 0].astype(jnp.int32)
    e1 = edges[..., 1].astype(jnp.int32)
    boff = (jnp.arange(B, dtype=jnp.int32) * N)[:, None]
    idx0 = (e0 + boff).reshape(BE)
    idx1 = (e1 + boff).reshape(BE)
    t = _make_gather(BE, F)(p1.reshape(B * N, F), p2.reshape(B * N, F),
                            idx0, idx1)

    wm, wm_t = _edge_stage(t, edge_features.reshape(BE, DE),
                     distance.reshape(BE, 1), w3,
                     b_msg.reshape(1, F), g_msg.reshape(1, F),
                     be_msg.reshape(1, F), sigma.reshape(1, 1),
                     beta_p.reshape(1, 1))

    bpc = B // NC
    locoff = ((jnp.arange(B, dtype=jnp.int32) % bpc) * N)[:, None]
    idx_sc = (e1 + locoff).reshape(BE)
    lane_off = jnp.arange(LANES, dtype=jnp.int32) * (bpc * N + 1)
    pos1d = (idx_sc[:, None] + lane_off[None, :]).reshape(BE * LANES)
    agg1d = _make_scatter(B, E, N, F)(wm_t, pos1d)
    agg_t = agg1d.reshape(NC, NS, LANES, bpc * N).transpose(1, 2, 0, 3)
    agg_t = agg_t.reshape(F, B * N)

    wcat = jnp.concatenate([Wq, Wk, Wv, Wg], axis=1)
    bcat = jnp.concatenate([bq, bk, bv, bg]).reshape(1, 4 * F)
    q, k, v, gt = _qkvg(nodes, agg_t, wcat[:F], wcat[F:], bcat)
    updated = _attention(q, k, v, gt, Wo, bo.reshape(1, F),
                         g_upd.reshape(1, F), be_upd.reshape(1, F), H)

    return (updated, wm.reshape(B, E, F), distance, edges)
